# Initial kernel scaffold; baseline (speedup 1.0000x reference)
import jax
import jax.numpy as jnp
from jax.experimental import pallas as pl


def kernel(x, tables):
    raise NotImplementedError("write your pallas kernel here")


# speedup vs baseline: 1.1508x; 1.1508x over previous
import functools

import jax
import jax.numpy as jnp
from jax import lax
from jax.experimental import pallas as pl
from jax.experimental.pallas import tpu as pltpu
from jax.experimental.pallas import tpu_sc as plsc

_FIELDS = 26
_VOCAB = 100000
_EMBED = 16
_BATCH = 16384
_BF = _BATCH * _FIELDS

_NC = 2
_NS = 16
_L = 16
_NW = _NC * _NS
_B_PER_W = _BF // _NW
_CHUNK = 3328
_NCHUNK = _B_PER_W // _CHUNK
_VPC = _CHUNK // _L

_mesh = plsc.VectorSubcoreMesh(core_axis_name="c", subcore_axis_name="s")


@functools.partial(
    pl.kernel,
    mesh=_mesh,
    out_type=jax.ShapeDtypeStruct((_BF, _EMBED), jnp.float32),
    scratch_types=[
        pltpu.VMEM((_CHUNK,), jnp.int32),
        pltpu.VMEM((_CHUNK, _EMBED), jnp.float32),
        pltpu.SemaphoreType.DMA,
    ],
    compiler_params=pltpu.CompilerParams(use_tc_tiling_on_sc=False),
)
def _gather_kernel(x_hbm, table_hbm, out_hbm, idx_v, rows_v, sem):
    wid = lax.axis_index("s") * _NC + lax.axis_index("c")
    base = wid * _B_PER_W

    def chunk_body(c, carry):
        start = base + c * _CHUNK
        pltpu.sync_copy(x_hbm.at[pl.ds(start, _CHUNK)], idx_v)

        def add_offsets(i, carry2):
            lane = lax.iota(jnp.int32, _L)
            row = start + i * _L + lane
            off = lax.rem(row, _FIELDS) * _VOCAB
            idx_v[pl.ds(i * _L, _L)] = idx_v[pl.ds(i * _L, _L)] + off
            return carry2

        lax.fori_loop(0, _VPC, add_offsets, 0)

        pltpu.async_copy(table_hbm.at[idx_v], rows_v, sem).wait()
        pltpu.sync_copy(rows_v, out_hbm.at[pl.ds(start, _CHUNK)])
        return carry

    lax.fori_loop(0, _NCHUNK, chunk_body, 0)


def kernel(x, tables):
    x_flat = x.reshape(_BF)
    tab_flat = tables.reshape(_FIELDS * _VOCAB, _EMBED)
    out = _gather_kernel(x_flat, tab_flat)
    return out.reshape(_BATCH, _FIELDS * _EMBED)


# speedup vs baseline: 6.4525x; 5.6071x over previous
import functools

import jax
import jax.numpy as jnp
from jax import lax
from jax.experimental import pallas as pl
from jax.experimental.pallas import tpu as pltpu
from jax.experimental.pallas import tpu_sc as plsc

_FIELDS = 26
_VOCAB = 100000
_EMBED = 16
_BATCH = 16384

_NC = 2
_NS = 16
_L = 16
_NW = _NC * _NS
_PAIRS = _FIELDS * _EMBED
_PPW = _PAIRS // _NW
_CB = 8192
_NB = _BATCH // _CB

_mesh = plsc.VectorSubcoreMesh(core_axis_name="c", subcore_axis_name="s")


@functools.partial(
    pl.kernel,
    mesh=_mesh,
    out_type=jax.ShapeDtypeStruct((_PAIRS, _BATCH), jnp.float32),
    scratch_types=[
        pltpu.VMEM((_VOCAB,), jnp.float32),
        pltpu.VMEM((_CB,), jnp.int32),
        pltpu.VMEM((_CB,), jnp.float32),
    ],
    compiler_params=pltpu.CompilerParams(needs_layout_passes=False),
)
def _gather_kernel(xt_hbm, tabt_hbm, outt_hbm, row_v, idx_v, out_v):
    wid = lax.axis_index("s") * _NC + lax.axis_index("c")

    def pair_body(k, carry):
        p = wid * _PPW + k
        f = p // _EMBED
        e = p % _EMBED
        pltpu.sync_copy(tabt_hbm.at[f, e], row_v)

        def chunk_body(c, carry2):
            b0 = c * _CB
            pltpu.sync_copy(xt_hbm.at[f, pl.ds(b0, _CB)], idx_v)

            def gather_body(j, carry3):
                sl = pl.ds(j * _L, _L)
                out_v[sl] = plsc.load_gather(row_v, [idx_v[sl]])
                return carry3

            lax.fori_loop(0, _CB // _L, gather_body, 0)
            pltpu.sync_copy(out_v, outt_hbm.at[p, pl.ds(b0, _CB)])
            return carry2

        lax.fori_loop(0, _NB, chunk_body, 0)
        return carry

    lax.fori_loop(0, _PPW, pair_body, 0)


def kernel(x, tables):
    xt = jnp.transpose(x)
    tabt = jnp.transpose(tables, (0, 2, 1))
    outt = _gather_kernel(xt, tabt)
    return jnp.transpose(outt)


# speedup vs baseline: 11.5730x; 1.7936x over previous
import functools

import jax
import jax.numpy as jnp
from jax import lax
from jax.experimental import pallas as pl
from jax.experimental.pallas import tpu as pltpu
from jax.experimental.pallas import tpu_sc as plsc

_FIELDS = 26
_VOCAB = 100000
_EMBED = 16
_BATCH = 16384

_NC = 2
_NS = 16
_L = 16
_NW = _NC * _NS
_PAIRS = _FIELDS * _EMBED
_PPW = _PAIRS // _NW
_CB = 4096
_NB = _BATCH // _CB

_mesh = plsc.VectorSubcoreMesh(core_axis_name="c", subcore_axis_name="s")


@functools.partial(
    pl.kernel,
    mesh=_mesh,
    out_type=jax.ShapeDtypeStruct((_PAIRS, _BATCH), jnp.float32),
    scratch_types=[
        pltpu.VMEM((_VOCAB,), jnp.float32),
        pltpu.VMEM((_BATCH,), jnp.int32),
        pltpu.VMEM((2, _CB), jnp.float32),
        pltpu.SemaphoreType.DMA,
        pltpu.SemaphoreType.DMA,
    ],
    compiler_params=pltpu.CompilerParams(needs_layout_passes=False),
)
def _gather_kernel(xt_hbm, tabt_hbm, outt_hbm, row_v, idx_v, out_v, sem0, sem1):
    wid = lax.axis_index("s") * _NC + lax.axis_index("c")
    sems = (sem0, sem1)

    def pair_body(k, prev_f):
        p = wid * _PPW + k
        f = p // _EMBED
        e = p % _EMBED

        @pl.when(f != prev_f)
        def _stage_idx():
            pltpu.sync_copy(xt_hbm.at[f], idx_v)

        pltpu.sync_copy(tabt_hbm.at[f, e], row_v)

        descs = [None, None]
        for c in range(_NB):
            buf = c % 2
            if descs[buf] is not None:
                descs[buf].wait()

            @plsc.parallel_loop(0, _CB // _L, unroll=8)
            def _gather(j):
                sl = pl.ds(j * _L, _L)
                out_v[buf, sl] = plsc.load_gather(
                    row_v, [idx_v[pl.ds(c * _CB + j * _L, _L)]]
                )

            descs[buf] = pltpu.async_copy(
                out_v.at[buf], outt_hbm.at[p, pl.ds(c * _CB, _CB)], sems[buf]
            )
        for d in descs:
            d.wait()
        return f

    lax.fori_loop(0, _PPW, pair_body, jnp.int32(-1))


def kernel(x, tables):
    xt = jnp.transpose(x)
    tabt = jnp.transpose(tables, (0, 2, 1))
    outt = _gather_kernel(xt, tabt)
    return jnp.transpose(outt)
